# baseline (speedup 1.0000x reference)
import functools

import jax
import jax.numpy as jnp
from jax import lax
from jax.experimental import pallas as pl
from jax.experimental.pallas import tpu as pltpu
from jax.experimental.pallas import tpu_sc as plsc

_LMBDA = 0.0001
_LANES = 16
_IDX_CHUNK = 128


def _sc_scores(pos_h, pos_t, pos_r, neg_h, neg_t, neg_r, ent, rel):
  B = pos_h.shape[0]
  H = ent.shape[1]
  info = plsc.get_sparse_core_info()
  nc, ns = info.num_cores, info.num_subcores
  nw = nc * ns
  bw = B // nw
  n_idx_chunks = bw // _IDX_CHUNK
  n_groups = bw // _LANES
  n_hchunks = H // _LANES

  mesh = plsc.VectorSubcoreMesh(core_axis_name="c", subcore_axis_name="s")

  @functools.partial(
      pl.kernel,
      out_type=(
          jax.ShapeDtypeStruct((B,), jnp.float32),
          jax.ShapeDtypeStruct((B,), jnp.float32),
          jax.ShapeDtypeStruct((nw, _LANES), jnp.float32),
      ),
      mesh=mesh,
      compiler_params=pltpu.CompilerParams(use_tc_tiling_on_sc=False),
      scratch_types=[
          pltpu.VMEM((bw,), jnp.int32),
          pltpu.VMEM((bw,), jnp.int32),
          pltpu.VMEM((bw,), jnp.int32),
          pltpu.VMEM((bw, H), jnp.float32),
          pltpu.VMEM((bw, H), jnp.float32),
          pltpu.VMEM((bw, H), jnp.float32),
          pltpu.VMEM((bw,), jnp.float32),
          pltpu.VMEM((_LANES,), jnp.float32),
          pltpu.SemaphoreType.DMA,
      ],
  )
  def k(ph, pt, pr, nh, nt, nr, ent_hbm, rel_hbm,
        ps_out, ns_out, reg_out,
        ih_v, it_v, ir_v, h_v, t_v, r_v, sc_v, acc_v, sem):
    wid = lax.axis_index("s") * nc + lax.axis_index("c")
    base = wid * bw
    lane = lax.iota(jnp.int32, _LANES)

    def phase(ih_hbm, it_hbm, ir_hbm, out_hbm, sq):
      pltpu.sync_copy(ih_hbm.at[pl.ds(base, bw)], ih_v)
      pltpu.sync_copy(it_hbm.at[pl.ds(base, bw)], it_v)
      pltpu.sync_copy(ir_hbm.at[pl.ds(base, bw)], ir_v)
      copies = []
      for j in range(n_idx_chunks):
        sl = pl.ds(j * _IDX_CHUNK, _IDX_CHUNK)
        copies.append(pltpu.async_copy(ent_hbm.at[ih_v.at[sl]], h_v.at[sl], sem))
        copies.append(pltpu.async_copy(ent_hbm.at[it_v.at[sl]], t_v.at[sl], sem))
        copies.append(pltpu.async_copy(rel_hbm.at[ir_v.at[sl]], r_v.at[sl], sem))
      for c in copies:
        c.wait()

      def group(g, sq):
        score_vec = jnp.zeros((_LANES,), jnp.float32)
        for j in range(_LANES):
          row = g * _LANES + j
          s = None
          for c in range(n_hchunks):
            sl = pl.ds(c * _LANES, _LANES)
            h = h_v[row, sl]
            t = t_v[row, sl]
            r = r_v[row, sl]
            p = h * r * t
            s = p if s is None else s + p
            sq = sq + (h * h + t * t + r * r)
          for sh in (8, 4, 2, 1):
            s = s + jnp.take(s, lane ^ sh)
          score_vec = jnp.where(lane == j, s, score_vec)
        sc_v[pl.ds(g * _LANES, _LANES)] = score_vec
        return sq

      sq = lax.fori_loop(0, n_groups, group, sq)
      pltpu.sync_copy(sc_v, out_hbm.at[pl.ds(base, bw)])
      return sq

    sq = jnp.zeros((_LANES,), jnp.float32)
    sq = phase(ph, pt, pr, ps_out, sq)
    sq = phase(nh, nt, nr, ns_out, sq)
    acc_v[...] = sq
    pltpu.sync_copy(acc_v, reg_out.at[wid])

  return k(pos_h, pos_t, pos_r, neg_h, neg_t, neg_r, ent, rel)


def _loss_body(p_ref, n_ref, py_ref, ny_ref, reg_ref, out_ref, *, B, H):
  xp = -py_ref[...] * p_ref[...]
  xn = -ny_ref[...] * n_ref[...]
  sp = (jnp.maximum(xp, 0.0) + jnp.log(1.0 + jnp.exp(-jnp.abs(xp)))
        + jnp.maximum(xn, 0.0) + jnp.log(1.0 + jnp.exp(-jnp.abs(xn))))
  reg = jnp.sum(reg_ref[...])
  out_ref[0, 0] = jnp.sum(sp) * (1.0 / B) + _LMBDA * reg * (1.0 / (B * H))


def kernel(pos_h, pos_t, pos_r, neg_h, neg_t, neg_r, pos_y, neg_y,
           ent_embeddings, rel_embeddings):
  B = pos_h.shape[0]
  H = ent_embeddings.shape[1]
  p_score, n_score, reg = _sc_scores(
      pos_h, pos_t, pos_r, neg_h, neg_t, neg_r, ent_embeddings, rel_embeddings)
  rows = B // 128
  out = pl.pallas_call(
      functools.partial(_loss_body, B=B, H=H),
      out_shape=jax.ShapeDtypeStruct((1, 1), jnp.float32),
      out_specs=pl.BlockSpec(memory_space=pltpu.SMEM),
  )(p_score.reshape(rows, 128), n_score.reshape(rows, 128),
    pos_y.reshape(rows, 128), neg_y.reshape(rows, 128), reg)
  return out[0, 0]


# speedup vs baseline: 1.0005x; 1.0005x over previous
import functools

import jax
import jax.numpy as jnp
from jax import lax
from jax.experimental import pallas as pl
from jax.experimental.pallas import tpu as pltpu
from jax.experimental.pallas import tpu_sc as plsc

_LMBDA = 0.0001
_LANES = 16


def _sc_scores(pos_h, pos_t, pos_r, neg_h, neg_t, neg_r, ent, rel):
  B = pos_h.shape[0]
  H = ent.shape[1]
  info = plsc.get_sparse_core_info()
  nc, ns = info.num_cores, info.num_subcores
  nw = nc * ns
  bw = B // nw
  n_groups = bw // _LANES
  n_hchunks = H // _LANES

  mesh = plsc.VectorSubcoreMesh(core_axis_name="c", subcore_axis_name="s")

  @functools.partial(
      pl.kernel,
      out_type=(
          jax.ShapeDtypeStruct((B,), jnp.float32),
          jax.ShapeDtypeStruct((B,), jnp.float32),
          jax.ShapeDtypeStruct((nw, _LANES), jnp.float32),
      ),
      mesh=mesh,
      compiler_params=pltpu.CompilerParams(use_tc_tiling_on_sc=False),
      scratch_types=[
          pltpu.VMEM((bw,), jnp.int32),
          pltpu.VMEM((bw,), jnp.int32),
          pltpu.VMEM((bw,), jnp.int32),
          pltpu.VMEM((bw, H), jnp.float32),
          pltpu.VMEM((bw, H), jnp.float32),
          pltpu.VMEM((bw, H), jnp.float32),
          pltpu.VMEM((bw,), jnp.float32),
          pltpu.VMEM((_LANES,), jnp.float32),
          pltpu.SemaphoreType.DMA,
      ],
  )
  def k(ph, pt, pr, nh, nt, nr, ent_hbm, rel_hbm,
        ps_out, ns_out, reg_out,
        ih_v, it_v, ir_v, h_v, t_v, r_v, sc_v, acc_v, sem):
    wid = lax.axis_index("s") * nc + lax.axis_index("c")
    base = wid * bw
    lane = lax.iota(jnp.int32, _LANES)

    def phase(ih_hbm, it_hbm, ir_hbm, out_hbm, sq):
      pltpu.sync_copy(ih_hbm.at[pl.ds(base, bw)], ih_v)
      pltpu.sync_copy(it_hbm.at[pl.ds(base, bw)], it_v)
      pltpu.sync_copy(ir_hbm.at[pl.ds(base, bw)], ir_v)
      cph = pltpu.async_copy(ent_hbm.at[ih_v], h_v, sem)
      cpt = pltpu.async_copy(ent_hbm.at[it_v], t_v, sem)
      cpr = pltpu.async_copy(rel_hbm.at[ir_v], r_v, sem)
      cph.wait()
      cpt.wait()
      cpr.wait()

      def group(g, sq):
        score_vec = jnp.zeros((_LANES,), jnp.float32)
        for j in range(_LANES):
          row = g * _LANES + j
          s = None
          for c in range(n_hchunks):
            sl = pl.ds(c * _LANES, _LANES)
            h = h_v[row, sl]
            t = t_v[row, sl]
            r = r_v[row, sl]
            p = h * r * t
            s = p if s is None else s + p
            sq = sq + (h * h + t * t + r * r)
          for sh in (8, 4, 2, 1):
            s = s + jnp.take(s, lane ^ sh)
          score_vec = jnp.where(lane == j, s, score_vec)
        sc_v[pl.ds(g * _LANES, _LANES)] = score_vec
        return sq

      sq = lax.fori_loop(0, n_groups, group, sq)
      pltpu.sync_copy(sc_v, out_hbm.at[pl.ds(base, bw)])
      return sq

    sq = jnp.zeros((_LANES,), jnp.float32)
    sq = phase(ph, pt, pr, ps_out, sq)
    sq = phase(nh, nt, nr, ns_out, sq)
    acc_v[...] = sq
    pltpu.sync_copy(acc_v, reg_out.at[wid])

  return k(pos_h, pos_t, pos_r, neg_h, neg_t, neg_r, ent, rel)


def _loss_body(p_ref, n_ref, py_ref, ny_ref, reg_ref, out_ref, *, B, H):
  xp = -py_ref[...] * p_ref[...]
  xn = -ny_ref[...] * n_ref[...]
  sp = (jnp.maximum(xp, 0.0) + jnp.log(1.0 + jnp.exp(-jnp.abs(xp)))
        + jnp.maximum(xn, 0.0) + jnp.log(1.0 + jnp.exp(-jnp.abs(xn))))
  reg = jnp.sum(reg_ref[...])
  out_ref[0, 0] = jnp.sum(sp) * (1.0 / B) + _LMBDA * reg * (1.0 / (B * H))


def kernel(pos_h, pos_t, pos_r, neg_h, neg_t, neg_r, pos_y, neg_y,
           ent_embeddings, rel_embeddings):
  B = pos_h.shape[0]
  H = ent_embeddings.shape[1]
  p_score, n_score, reg = _sc_scores(
      pos_h, pos_t, pos_r, neg_h, neg_t, neg_r, ent_embeddings, rel_embeddings)
  rows = B // 128
  out = pl.pallas_call(
      functools.partial(_loss_body, B=B, H=H),
      out_shape=jax.ShapeDtypeStruct((1, 1), jnp.float32),
      out_specs=pl.BlockSpec(memory_space=pltpu.SMEM),
  )(p_score.reshape(rows, 128), n_score.reshape(rows, 128),
    pos_y.reshape(rows, 128), neg_y.reshape(rows, 128), reg)
  return out[0, 0]
